# baseline (speedup 1.0000x reference)
import jax
import jax.numpy as jnp
from jax.experimental import pallas as pl
from jax.experimental.pallas import tpu as pltpu

_EMB_START = 64
_EMB_END = 192
_BLOCK_ROWS = 2000


def _fused_block(mol_ref, nf_ref, wm_ref, wd_ref, out_ref):
    emb16 = jnp.tanh(
        jnp.dot(
            mol_ref[...].astype(jnp.bfloat16),
            wm_ref[...].astype(jnp.bfloat16),
            preferred_element_type=jnp.float32,
        ).astype(jnp.bfloat16)
    )
    nf = nf_ref[...].astype(jnp.bfloat16)
    spliced = jnp.concatenate(
        [nf[:, :_EMB_START], emb16, nf[:, _EMB_END:]], axis=1)
    acc16 = jnp.dot(spliced, wd_ref[...].astype(jnp.bfloat16),
                    preferred_element_type=jnp.float32).astype(jnp.bfloat16)
    out_ref[...] = jnp.tanh(acc16).astype(jnp.float32)


def kernel(molecules, nodes_features, type_mask0, type_mask2, W_mol, W_drug):
    del type_mask0, type_mask2
    n, d_feat = nodes_features.shape
    mol_feat = molecules.shape[1]
    b = _BLOCK_ROWS
    return pl.pallas_call(
        _fused_block,
        grid=(n // b,),
        in_specs=[
            pl.BlockSpec((b, mol_feat), lambda i: (i, 0)),
            pl.BlockSpec((b, d_feat), lambda i: (i, 0)),
            pl.BlockSpec(W_mol.shape, lambda i: (0, 0)),
            pl.BlockSpec(W_drug.shape, lambda i: (0, 0)),
        ],
        out_specs=pl.BlockSpec((b, d_feat), lambda i: (i, 0)),
        out_shape=jax.ShapeDtypeStruct((n, d_feat), nodes_features.dtype),
        compiler_params=pltpu.CompilerParams(
            dimension_semantics=("arbitrary",),
        ),
    )(molecules, nodes_features, W_mol, W_drug)


# speedup vs baseline: 1.1447x; 1.1447x over previous
import jax
import jax.numpy as jnp
from jax.experimental import pallas as pl
from jax.experimental.pallas import tpu as pltpu

_EMB_START = 64
_EMB_END = 192
_BLOCK_ROWS = 10000


def _fused_block(mol_ref, nf_ref, wm_ref, wd_ref, out_ref):
    emb16 = jnp.tanh(
        jnp.dot(
            mol_ref[...].astype(jnp.bfloat16),
            wm_ref[...].astype(jnp.bfloat16),
            preferred_element_type=jnp.float32,
        ).astype(jnp.bfloat16)
    )
    nf = nf_ref[...].astype(jnp.bfloat16)
    spliced = jnp.concatenate(
        [nf[:, :_EMB_START], emb16, nf[:, _EMB_END:]], axis=1)
    acc16 = jnp.dot(spliced, wd_ref[...].astype(jnp.bfloat16),
                    preferred_element_type=jnp.float32).astype(jnp.bfloat16)
    out_ref[...] = jnp.tanh(acc16).astype(jnp.float32)


def kernel(molecules, nodes_features, type_mask0, type_mask2, W_mol, W_drug):
    del type_mask0, type_mask2
    n, d_feat = nodes_features.shape
    mol_feat = molecules.shape[1]
    b = _BLOCK_ROWS
    return pl.pallas_call(
        _fused_block,
        grid=(n // b,),
        in_specs=[
            pl.BlockSpec((b, mol_feat), lambda i: (i, 0)),
            pl.BlockSpec((b, d_feat), lambda i: (i, 0)),
            pl.BlockSpec(W_mol.shape, lambda i: (0, 0)),
            pl.BlockSpec(W_drug.shape, lambda i: (0, 0)),
        ],
        out_specs=pl.BlockSpec((b, d_feat), lambda i: (i, 0)),
        out_shape=jax.ShapeDtypeStruct((n, d_feat), nodes_features.dtype),
        compiler_params=pltpu.CompilerParams(
            dimension_semantics=("arbitrary",),
        ),
    )(molecules, nodes_features, W_mol, W_drug)
